# baseline (speedup 1.0000x reference)
import functools

import jax
import jax.numpy as jnp
from jax import lax
from jax.experimental import pallas as pl
from jax.experimental.pallas import tpu as pltpu
from jax.experimental.pallas import tpu_sc as plsc

_L = 16


@functools.cache
def _build(B, C, POOL, TOK, CH):
    info = plsc.get_sparse_core_info()
    NC, NS = info.num_cores, info.num_subcores
    NW = NC * NS
    nsel = C - 1
    nslots = nsel * TOK
    b_per_sub = B // NS
    BK = 128
    nbk = B // BK
    slots_per_w = (nslots + NW - 1) // NW
    assert B % BK == 0 and B % NS == 0 and CH % _L == 0
    assert _L <= nsel <= 2 * _L and C >= _L + 1

    mesh = plsc.VectorSubcoreMesh(core_axis_name="c", subcore_axis_name="s")

    @functools.partial(
        pl.kernel,
        out_type=jax.ShapeDtypeStruct((nslots, B, CH), jnp.float32),
        mesh=mesh,
        compiler_params=pltpu.CompilerParams(needs_layout_passes=False),
        scratch_types=[
            pltpu.VMEM((b_per_sub, C), jnp.int32),
            pltpu.VMEM((nsel, b_per_sub), jnp.int32),
            pltpu.VMEM((B,), jnp.int32),
            pltpu.VMEM((B,), jnp.int32),
            pltpu.VMEM((4 * BK, CH), jnp.float32),
            pltpu.VMEM_SHARED((nsel * B,), jnp.int32),
            pltpu.SemaphoreType.DMA,
            pltpu.SemaphoreType.DMA,
            pltpu.SemaphoreType.DMA,
        ],
    )
    def kfn(lp_hbm, table_hbm, out_hbm,
            lp_v, idxt_v, row_v, gidx_v, bigbuf, sharedT,
            sem0, sem1, semp):
        cid = lax.axis_index("c")
        sid = lax.axis_index("s")
        wid = sid * NC + cid

        b0 = sid * b_per_sub
        pltpu.sync_copy(lp_hbm.at[pl.ds(b0, b_per_sub)], lp_v)

        iota = lax.iota(jnp.int32, _L)
        zeros = iota * 0
        ones = zeros + 1
        fill = zeros + TOK
        for s in range(nsel):
            for c in range(b_per_sub // _L):
                idxt_v[s, pl.ds(c * _L, _L)] = fill

        v0 = (iota + 1) * TOK
        v1 = (iota + (C - _L)) * TOK
        tailmask = iota >= (2 * _L - nsel)
        for bb in range(b_per_sub):
            ch0 = lp_v[bb, pl.ds(1, _L)]
            ch1 = lp_v[bb, pl.ds(C - _L, _L)]
            m0 = ch0 != zeros
            m1 = (ch1 != zeros) & tailmask
            m0i = jnp.where(m0, ones, zeros)
            m1i = jnp.where(m1, ones, zeros)
            c0 = plsc.cumsum(m0i)
            n0 = jnp.sum(m0i)
            n0v = lax.broadcast_in_dim(n0, (_L,), ())
            c1 = plsc.cumsum(m1i)
            bbv = zeros + bb
            plsc.store_scatter(idxt_v, [c0 - 1, bbv], v0, mask=m0)
            plsc.store_scatter(idxt_v, [c1 + n0v - 1, bbv], v1, mask=m1)

        handles = [
            pltpu.async_copy(idxt_v.at[s],
                             sharedT.at[pl.ds(s * B + b0, b_per_sub)], semp)
            for s in range(nsel)
        ]
        for h in handles:
            h.wait()
        plsc.subcore_barrier()

        @pl.loop(0, slots_per_w)
        def _slot(j):
            slot = wid + j * NW

            @pl.when(slot < nslots)
            def _():
                s = slot // TOK
                t = slot % TOK
                pltpu.sync_copy(sharedT.at[pl.ds(s * B, B)], row_v)
                tv = lax.broadcast_in_dim(t, (_L,), ())
                v0 = plsc.load_gather(row_v, [zeros]) + tv

                acc = zeros == zeros
                for c in range(B // _L):
                    acc = acc & (row_v[pl.ds(c * _L, _L)] + tv == v0)
                uniform = jnp.all(acc)

                @pl.when(uniform)
                def _fast():
                    q = jnp.max(v0)
                    pltpu.sync_copy(table_hbm.at[pl.ds(q, 1)],
                                    bigbuf.at[pl.ds(0, 1)])
                    regs = [bigbuf[0, pl.ds(i * _L, _L)]
                            for i in range(CH // _L)]
                    for r in range(1, BK):
                        for i in range(CH // _L):
                            bigbuf[r, pl.ds(i * _L, _L)] = regs[i]
                    hs = [
                        pltpu.async_copy(
                            bigbuf.at[pl.ds(0, BK)],
                            out_hbm.at[slot, pl.ds(k * BK, BK)], sem1)
                        for k in range(nbk)
                    ]
                    for h in hs:
                        h.wait()

                @pl.when(jnp.logical_not(uniform))
                def _general():
                    for c in range(B // _L):
                        gidx_v[pl.ds(c * _L, _L)] = (
                            row_v[pl.ds(c * _L, _L)] + tv)

                    def gather(k, buf, sem):
                        return pltpu.async_copy(
                            table_hbm.at[gidx_v.at[pl.ds(k * BK, BK)]],
                            buf, sem)

                    bufs = (bigbuf.at[pl.ds(0, BK)], bigbuf.at[pl.ds(BK, BK)])
                    sems = (sem0, sem1)
                    pending = gather(0, bufs[0], sem0)
                    for k in range(nbk):
                        nxt = (gather(k + 1, bufs[(k + 1) % 2],
                                      sems[(k + 1) % 2])
                               if k + 1 < nbk else None)
                        pending.wait()
                        pltpu.sync_copy(
                            bufs[k % 2], out_hbm.at[slot, pl.ds(k * BK, BK)])
                        pending = nxt

    return kfn


def kernel(label_indices, tokens):
    B, C = label_indices.shape
    POOL, TOK, CH = tokens.shape
    table = tokens.reshape(POOL * TOK, CH)
    out = _build(B, C, POOL, TOK, CH)(label_indices, table)
    return jnp.transpose(out, (1, 0, 2))


# speedup vs baseline: 1.0865x; 1.0865x over previous
import functools

import jax
import jax.numpy as jnp
from jax import lax
from jax.experimental import pallas as pl
from jax.experimental.pallas import tpu as pltpu
from jax.experimental.pallas import tpu_sc as plsc

_L = 16


@functools.cache
def _build(B, C, POOL, TOK, CH):
    info = plsc.get_sparse_core_info()
    NC, NS = info.num_cores, info.num_subcores
    NW = NC * NS
    nsel = C - 1
    nslots = nsel * TOK
    b_per_sub = B // NS
    BK = 128
    nbk = B // BK
    slots_per_w = (nslots + NW - 1) // NW
    assert B % BK == 0 and B % NS == 0 and CH % _L == 0
    assert _L <= nsel <= 2 * _L and C >= _L + 1

    mesh = plsc.VectorSubcoreMesh(core_axis_name="c", subcore_axis_name="s")

    @functools.partial(
        pl.kernel,
        out_type=jax.ShapeDtypeStruct((nslots, B, CH), jnp.float32),
        mesh=mesh,
        compiler_params=pltpu.CompilerParams(needs_layout_passes=False),
        scratch_types=[
            pltpu.VMEM((b_per_sub, C), jnp.int32),
            pltpu.VMEM((nsel, b_per_sub), jnp.int32),
            pltpu.VMEM((B,), jnp.int32),
            pltpu.VMEM((4 * BK, CH), jnp.float32),
            pltpu.VMEM((_L, TOK, CH), jnp.float32),
            pltpu.VMEM_SHARED((nsel * B,), jnp.int32),
            pltpu.SemaphoreType.DMA,
            pltpu.SemaphoreType.DMA,
            pltpu.SemaphoreType.DMA,
        ],
    )
    def kfn(lp_hbm, table_hbm, out_hbm,
            lp_v, idxt_v, row_v, bigbuf, rowbuf, sharedT,
            sem0, sem1, semp):
        cid = lax.axis_index("c")
        sid = lax.axis_index("s")
        wid = sid * NC + cid

        b0 = sid * b_per_sub
        pltpu.sync_copy(lp_hbm.at[pl.ds(b0, b_per_sub)], lp_v)

        iota = lax.iota(jnp.int32, _L)
        zeros = iota * 0
        ones = zeros + 1
        fill = zeros + TOK
        for s in range(nsel):
            for c in range(b_per_sub // _L):
                idxt_v[s, pl.ds(c * _L, _L)] = fill

        v0 = (iota + 1) * TOK
        v1 = (iota + (C - _L)) * TOK
        tailmask = iota >= (2 * _L - nsel)
        for bb in range(b_per_sub):
            ch0 = lp_v[bb, pl.ds(1, _L)]
            ch1 = lp_v[bb, pl.ds(C - _L, _L)]
            m0 = ch0 != zeros
            m1 = (ch1 != zeros) & tailmask
            m0i = jnp.where(m0, ones, zeros)
            m1i = jnp.where(m1, ones, zeros)
            c0 = plsc.cumsum(m0i)
            n0 = jnp.sum(m0i)
            n0v = lax.broadcast_in_dim(n0, (_L,), ())
            c1 = plsc.cumsum(m1i)
            bbv = zeros + bb
            plsc.store_scatter(idxt_v, [c0 - 1, bbv], v0, mask=m0)
            plsc.store_scatter(idxt_v, [c1 + n0v - 1, bbv], v1, mask=m1)

        handles = [
            pltpu.async_copy(idxt_v.at[s],
                             sharedT.at[pl.ds(s * B + b0, b_per_sub)], semp)
            for s in range(nsel)
        ]
        for h in handles:
            h.wait()
        plsc.subcore_barrier()

        @pl.loop(0, slots_per_w)
        def _slot(j):
            slot = wid + j * NW

            @pl.when(slot < nslots)
            def _():
                s = slot // TOK
                t = slot % TOK
                pltpu.sync_copy(sharedT.at[pl.ds(s * B, B)], row_v)
                tv = lax.broadcast_in_dim(t, (_L,), ())
                v0 = plsc.load_gather(row_v, [zeros]) + tv

                acc = zeros == zeros
                for c in range(B // _L):
                    acc = acc & (row_v[pl.ds(c * _L, _L)] + tv == v0)
                uniform = jnp.all(acc)

                @pl.when(uniform)
                def _fast():
                    q = jnp.max(v0)
                    pltpu.sync_copy(
                        table_hbm.at[q // TOK, pl.ds(q % TOK, 1)],
                        bigbuf.at[pl.ds(0, 1)])
                    regs = [bigbuf[0, pl.ds(i * _L, _L)]
                            for i in range(CH // _L)]
                    for r in range(1, BK):
                        for i in range(CH // _L):
                            bigbuf[r, pl.ds(i * _L, _L)] = regs[i]
                    hs = [
                        pltpu.async_copy(
                            bigbuf.at[pl.ds(0, BK)],
                            out_hbm.at[slot, pl.ds(k * BK, BK)], sem1)
                        for k in range(nbk)
                    ]
                    for h in hs:
                        h.wait()

                @pl.when(jnp.logical_not(uniform))
                def _general():
                    @pl.loop(0, B // _L)
                    def _chunk(c):
                        rows = row_v[pl.ds(c * _L, _L)] // TOK
                        pltpu.async_copy(table_hbm.at[rows], rowbuf,
                                         sem0).wait()
                        hs = [
                            pltpu.async_copy(
                                rowbuf.at[i, pl.ds(t, 1)],
                                out_hbm.at[slot, pl.ds(c * _L + i, 1)],
                                sem1)
                            for i in range(_L)
                        ]
                        for h in hs:
                            h.wait()

    return kfn


def kernel(label_indices, tokens):
    B, C = label_indices.shape
    POOL, TOK, CH = tokens.shape
    out = _build(B, C, POOL, TOK, CH)(label_indices, tokens)
    return jnp.transpose(out, (1, 0, 2))


# speedup vs baseline: 1.1121x; 1.0236x over previous
import functools

import jax
import jax.numpy as jnp
from jax import lax
from jax.experimental import pallas as pl
from jax.experimental.pallas import tpu as pltpu
from jax.experimental.pallas import tpu_sc as plsc

_L = 16


@functools.cache
def _build(B, C, POOL, TOK, CH):
    info = plsc.get_sparse_core_info()
    NC, NS = info.num_cores, info.num_subcores
    NW = NC * NS
    nsel = C - 1
    nslots = nsel * TOK
    b_per_sub = B // NS
    BK = 128
    nbk = B // BK
    slots_per_w = (nslots + NW - 1) // NW
    assert B % BK == 0 and B % NS == 0 and CH % _L == 0
    assert _L <= nsel <= 2 * _L and C >= _L + 1

    mesh = plsc.VectorSubcoreMesh(core_axis_name="c", subcore_axis_name="s")

    @functools.partial(
        pl.kernel,
        out_type=jax.ShapeDtypeStruct((nslots, B, CH), jnp.float32),
        mesh=mesh,
        compiler_params=pltpu.CompilerParams(needs_layout_passes=False),
        scratch_types=[
            pltpu.VMEM((b_per_sub, C), jnp.int32),
            pltpu.VMEM((nsel, b_per_sub), jnp.int32),
            pltpu.VMEM((B,), jnp.int32),
            pltpu.VMEM((BK, CH), jnp.float32),
            pltpu.VMEM((_L, TOK, CH), jnp.float32),
            pltpu.VMEM_SHARED((nsel * B,), jnp.int32),
            pltpu.SemaphoreType.DMA,
            pltpu.SemaphoreType.DMA,
            pltpu.SemaphoreType.DMA,
        ],
    )
    def kfn(lp_hbm, table_hbm, out_hbm,
            lp_v, idxt_v, row_v, bigbuf, rowbuf, sharedT,
            sem0, sem1, semp):
        cid = lax.axis_index("c")
        sid = lax.axis_index("s")
        wid = sid * NC + cid

        b0 = sid * b_per_sub
        pltpu.sync_copy(lp_hbm.at[pl.ds(b0, b_per_sub)], lp_v)

        iota = lax.iota(jnp.int32, _L)
        zeros = iota * 0
        ones = zeros + 1
        fill = zeros + TOK
        for s in range(nsel):
            for c in range(b_per_sub // _L):
                idxt_v[s, pl.ds(c * _L, _L)] = fill

        v0 = (iota + 1) * TOK
        v1 = (iota + (C - _L)) * TOK
        tailmask = iota >= (2 * _L - nsel)
        for bb in range(b_per_sub):
            ch0 = lp_v[bb, pl.ds(1, _L)]
            ch1 = lp_v[bb, pl.ds(C - _L, _L)]
            m0 = ch0 != zeros
            m1 = (ch1 != zeros) & tailmask
            m0i = jnp.where(m0, ones, zeros)
            m1i = jnp.where(m1, ones, zeros)
            c0 = plsc.cumsum(m0i)
            n0 = jnp.sum(m0i)
            n0v = lax.broadcast_in_dim(n0, (_L,), ())
            c1 = plsc.cumsum(m1i)
            bbv = zeros + bb
            plsc.store_scatter(idxt_v, [c0 - 1, bbv], v0, mask=m0)
            plsc.store_scatter(idxt_v, [c1 + n0v - 1, bbv], v1, mask=m1)

        handles = [
            pltpu.async_copy(idxt_v.at[s],
                             sharedT.at[pl.ds(s * B + b0, b_per_sub)], semp)
            for s in range(nsel)
        ]
        for h in handles:
            h.wait()
        plsc.subcore_barrier()

        @pl.loop(0, slots_per_w, init_carry=jnp.all(zeros == zeros))
        def all_u(j, acc_all):
            slot_raw = wid + j * NW
            slot = jnp.minimum(slot_raw, nslots - 1)
            s = slot // TOK
            t = slot % TOK
            pltpu.sync_copy(sharedT.at[pl.ds(s * B, B)], row_v)
            tv = lax.broadcast_in_dim(t, (_L,), ())
            v0 = plsc.load_gather(row_v, [zeros]) + tv
            acc = zeros == zeros
            for c in range(B // _L):
                acc = acc & (row_v[pl.ds(c * _L, _L)] + tv == v0)
            slot_ok = jnp.all(acc) | (slot_raw >= nslots)
            return acc_all & slot_ok

        nfull = nslots // NW
        nrem = nslots - nfull * NW

        RB = 32
        def fetch_replicate(slot, half):
            s = slot // TOK
            t = slot % TOK
            pltpu.sync_copy(sharedT.at[pl.ds(s * B, B)], row_v)
            v0 = plsc.load_gather(row_v, [zeros])
            q = jnp.max(v0) + t
            pltpu.sync_copy(table_hbm.at[q // TOK, pl.ds(q % TOK, 1)],
                            half.at[pl.ds(0, 1)])
            regs = [half[0, pl.ds(i * _L, _L)] for i in range(CH // _L)]
            for r in range(1, RB):
                for i in range(CH // _L):
                    half[r, pl.ds(i * _L, _L)] = regs[i]

        @pl.when(all_u)
        def _fast_all():
            halves = (bigbuf.at[pl.ds(0, RB)], bigbuf.at[pl.ds(RB, RB)])
            sems = (sem0, sem1)
            pending = [[], []]
            for j in range(nfull):
                p = j % 2
                for h in pending[p]:
                    h.wait()
                slot = wid + j * NW
                fetch_replicate(slot, halves[p])
                pending[p] = [
                    pltpu.async_copy(
                        halves[p],
                        out_hbm.at[slot, pl.ds(k * RB, RB)], sems[p])
                    for k in range(B // RB)
                ]
            for hs in pending:
                for h in hs:
                    h.wait()

            @pl.when(wid < nrem)
            def _rem():
                slot = wid + nfull * NW
                fetch_replicate(slot, halves[0])
                hs = [
                    pltpu.async_copy(
                        halves[0],
                        out_hbm.at[slot, pl.ds(k * RB, RB)], sem0)
                    for k in range(B // RB)
                ]
                for h in hs:
                    h.wait()

        @pl.when(jnp.logical_not(all_u))
        def _slow_all():
            @pl.loop(0, slots_per_w)
            def _slot(j):
                slot = wid + j * NW

                @pl.when(slot < nslots)
                def _():
                    t = slot % TOK
                    tv = lax.broadcast_in_dim(t, (_L,), ())
                    pltpu.sync_copy(
                        sharedT.at[pl.ds((slot // TOK) * B, B)], row_v)
                    v0 = plsc.load_gather(row_v, [zeros]) + tv
                    acc = zeros == zeros
                    for c in range(B // _L):
                        acc = acc & (row_v[pl.ds(c * _L, _L)] + tv == v0)
                    uniform = jnp.all(acc)

                    @pl.when(uniform)
                    def _fast():
                        fetch_replicate(slot, bigbuf.at[pl.ds(0, RB)])
                        hs = [
                            pltpu.async_copy(
                                bigbuf.at[pl.ds(0, RB)],
                                out_hbm.at[slot, pl.ds(k * RB, RB)], sem1)
                            for k in range(B // RB)
                        ]
                        for h in hs:
                            h.wait()

                    @pl.when(jnp.logical_not(uniform))
                    def _general():
                        @pl.loop(0, B // _L)
                        def _chunk(c):
                            rows = row_v[pl.ds(c * _L, _L)] // TOK
                            pltpu.async_copy(table_hbm.at[rows], rowbuf,
                                             sem0).wait()
                            hs = [
                                pltpu.async_copy(
                                    rowbuf.at[i, pl.ds(t, 1)],
                                    out_hbm.at[slot,
                                               pl.ds(c * _L + i, 1)],
                                    sem1)
                                for i in range(_L)
                            ]
                            for h in hs:
                                h.wait()

    return kfn


def kernel(label_indices, tokens):
    B, C = label_indices.shape
    POOL, TOK, CH = tokens.shape
    out = _build(B, C, POOL, TOK, CH)(label_indices, tokens)
    return jnp.transpose(out, (1, 0, 2))


# speedup vs baseline: 1.1170x; 1.0044x over previous
import functools

import jax
import jax.numpy as jnp
from jax import lax
from jax.experimental import pallas as pl
from jax.experimental.pallas import tpu as pltpu
from jax.experimental.pallas import tpu_sc as plsc

_L = 16


@functools.cache
def _build(B, C, POOL, TOK, CH):
    info = plsc.get_sparse_core_info()
    NC, NS = info.num_cores, info.num_subcores
    NW = NC * NS
    nsel = C - 1
    nslots = nsel * TOK
    b_per_sub = B // NS
    BK = 128
    nbk = B // BK
    slots_per_w = (nslots + NW - 1) // NW
    assert B % BK == 0 and B % NS == 0 and CH % _L == 0
    assert _L <= nsel <= 2 * _L and C >= _L + 1

    mesh = plsc.VectorSubcoreMesh(core_axis_name="c", subcore_axis_name="s")

    @functools.partial(
        pl.kernel,
        out_type=jax.ShapeDtypeStruct((nslots, B, CH), jnp.float32),
        mesh=mesh,
        compiler_params=pltpu.CompilerParams(needs_layout_passes=False),
        scratch_types=[
            pltpu.VMEM((b_per_sub, C), jnp.int32),
            pltpu.VMEM((nsel, b_per_sub), jnp.int32),
            pltpu.VMEM((B,), jnp.int32),
            pltpu.VMEM((BK, CH), jnp.float32),
            pltpu.VMEM((_L, TOK, CH), jnp.float32),
            pltpu.VMEM_SHARED((nsel * B,), jnp.int32),
            pltpu.SemaphoreType.DMA,
            pltpu.SemaphoreType.DMA,
            pltpu.SemaphoreType.DMA,
        ],
    )
    def kfn(lp_hbm, table_hbm, out_hbm,
            lp_v, idxt_v, row_v, bigbuf, rowbuf, sharedT,
            sem0, sem1, semp):
        cid = lax.axis_index("c")
        sid = lax.axis_index("s")
        wid = sid * NC + cid

        b0 = sid * b_per_sub
        pltpu.sync_copy(lp_hbm.at[pl.ds(b0, b_per_sub)], lp_v)

        iota = lax.iota(jnp.int32, _L)
        zeros = iota * 0
        ones = zeros + 1
        fill = zeros + TOK
        for s in range(nsel):
            for c in range(b_per_sub // _L):
                idxt_v[s, pl.ds(c * _L, _L)] = fill

        v0 = (iota + 1) * TOK
        v1 = (iota + (C - _L)) * TOK
        tailmask = iota >= (2 * _L - nsel)
        for bb in range(b_per_sub):
            ch0 = lp_v[bb, pl.ds(1, _L)]
            ch1 = lp_v[bb, pl.ds(C - _L, _L)]
            m0 = ch0 != zeros
            m1 = (ch1 != zeros) & tailmask
            m0i = jnp.where(m0, ones, zeros)
            m1i = jnp.where(m1, ones, zeros)
            c0 = plsc.cumsum(m0i)
            n0 = jnp.sum(m0i)
            n0v = lax.broadcast_in_dim(n0, (_L,), ())
            c1 = plsc.cumsum(m1i)
            bbv = zeros + bb
            plsc.store_scatter(idxt_v, [c0 - 1, bbv], v0, mask=m0)
            plsc.store_scatter(idxt_v, [c1 + n0v - 1, bbv], v1, mask=m1)

        handles = [
            pltpu.async_copy(idxt_v.at[s],
                             sharedT.at[pl.ds(s * B + b0, b_per_sub)], semp)
            for s in range(nsel)
        ]
        for h in handles:
            h.wait()
        plsc.subcore_barrier()

        @pl.loop(0, slots_per_w, init_carry=jnp.all(zeros == zeros))
        def all_u(j, acc_all):
            slot_raw = wid + j * NW
            slot = jnp.minimum(slot_raw, nslots - 1)
            s = slot // TOK
            t = slot % TOK
            pltpu.sync_copy(sharedT.at[pl.ds(s * B, B)], row_v)
            tv = lax.broadcast_in_dim(t, (_L,), ())
            v0 = plsc.load_gather(row_v, [zeros]) + tv
            acc = zeros == zeros
            for c in range(B // _L):
                acc = acc & (row_v[pl.ds(c * _L, _L)] + tv == v0)
            slot_ok = jnp.all(acc) | (slot_raw >= nslots)
            return acc_all & slot_ok

        nfull = nslots // NW
        nrem = nslots - nfull * NW

        RB = 32
        def slot_q(slot):
            s = slot // TOK
            t = slot % TOK
            pltpu.sync_copy(sharedT.at[pl.ds(s * B, B)], row_v)
            v0 = plsc.load_gather(row_v, [zeros])
            return jnp.max(v0) + t

        def fetch_replicate_q(q, half):
            pltpu.sync_copy(table_hbm.at[q // TOK, pl.ds(q % TOK, 1)],
                            half.at[pl.ds(0, 1)])
            regs = [half[0, pl.ds(i * _L, _L)] for i in range(CH // _L)]
            for r in range(1, RB):
                for i in range(CH // _L):
                    half[r, pl.ds(i * _L, _L)] = regs[i]

        def fetch_replicate(slot, half):
            fetch_replicate_q(slot_q(slot), half)

        @pl.when(all_u)
        def _fast_all():
            halves = (bigbuf.at[pl.ds(0, RB)], bigbuf.at[pl.ds(RB, RB)])
            sems = (sem0, sem1)
            pending = [[], []]
            for j in range(nfull):
                p = j % 2
                slot = wid + j * NW
                q = slot_q(slot)
                for h in pending[p]:
                    h.wait()
                fetch_replicate_q(q, halves[p])
                pending[p] = [
                    pltpu.async_copy(
                        halves[p],
                        out_hbm.at[slot, pl.ds(k * RB, RB)], sems[p])
                    for k in range(B // RB)
                ]
            for hs in pending:
                for h in hs:
                    h.wait()

            @pl.when(wid < nrem)
            def _rem():
                slot = wid + nfull * NW
                fetch_replicate(slot, halves[0])
                hs = [
                    pltpu.async_copy(
                        halves[0],
                        out_hbm.at[slot, pl.ds(k * RB, RB)], sem0)
                    for k in range(B // RB)
                ]
                for h in hs:
                    h.wait()

        @pl.when(jnp.logical_not(all_u))
        def _slow_all():
            @pl.loop(0, slots_per_w)
            def _slot(j):
                slot = wid + j * NW

                @pl.when(slot < nslots)
                def _():
                    t = slot % TOK
                    tv = lax.broadcast_in_dim(t, (_L,), ())
                    pltpu.sync_copy(
                        sharedT.at[pl.ds((slot // TOK) * B, B)], row_v)
                    v0 = plsc.load_gather(row_v, [zeros]) + tv
                    acc = zeros == zeros
                    for c in range(B // _L):
                        acc = acc & (row_v[pl.ds(c * _L, _L)] + tv == v0)
                    uniform = jnp.all(acc)

                    @pl.when(uniform)
                    def _fast():
                        fetch_replicate(slot, bigbuf.at[pl.ds(0, RB)])
                        hs = [
                            pltpu.async_copy(
                                bigbuf.at[pl.ds(0, RB)],
                                out_hbm.at[slot, pl.ds(k * RB, RB)], sem1)
                            for k in range(B // RB)
                        ]
                        for h in hs:
                            h.wait()

                    @pl.when(jnp.logical_not(uniform))
                    def _general():
                        @pl.loop(0, B // _L)
                        def _chunk(c):
                            rows = row_v[pl.ds(c * _L, _L)] // TOK
                            pltpu.async_copy(table_hbm.at[rows], rowbuf,
                                             sem0).wait()
                            hs = [
                                pltpu.async_copy(
                                    rowbuf.at[i, pl.ds(t, 1)],
                                    out_hbm.at[slot,
                                               pl.ds(c * _L + i, 1)],
                                    sem1)
                                for i in range(_L)
                            ]
                            for h in hs:
                                h.wait()

    return kfn


def kernel(label_indices, tokens):
    B, C = label_indices.shape
    POOL, TOK, CH = tokens.shape
    out = _build(B, C, POOL, TOK, CH)(label_indices, tokens)
    return jnp.transpose(out, (1, 0, 2))
